# initial kernel scaffold (unmeasured)
import functools

import jax
import jax.numpy as jnp
from jax import lax
from jax.experimental import pallas as pl
from jax.experimental.pallas import tpu as pltpu

N_DEV = 4


def kernel(x, router_W, route_idx, expert_W):
    del router_W
    n_tok, d_model = x.shape
    e_local, _, d_ff = expert_W.shape

    def body(x_ref, idx_ref, w_ref, out_ref, comm_ref, send_sems, recv_sems):
        my_pos = lax.axis_index("i")
        left = (my_pos - 1) % N_DEV
        right = (my_pos + 1) % N_DEV

        route = idx_ref[:, :]
        acc = jnp.zeros((n_tok, d_ff), jnp.float32)
        for el in range(e_local):
            ge = my_pos * e_local + el
            mask = (route == ge).astype(jnp.float32)
            xm = x_ref[:, :] * mask
            acc = acc + jnp.dot(
                xm, w_ref[el], preferred_element_type=jnp.float32
            )
        out_ref[:, :] = acc
        comm_ref[0, :, :] = acc

        barrier_sem = pltpu.get_barrier_semaphore()
        for nbr in [left, right]:
            pl.semaphore_signal(
                barrier_sem, inc=1,
                device_id=(nbr,), device_id_type=pl.DeviceIdType.MESH,
            )
        pl.semaphore_wait(barrier_sem, 2)

        for h in range(N_DEV - 1):
            rdma = pltpu.make_async_remote_copy(
                src_ref=comm_ref.at[h],
                dst_ref=comm_ref.at[h + 1],
                send_sem=send_sems.at[h],
                recv_sem=recv_sems.at[h],
                device_id=(right,),
                device_id_type=pl.DeviceIdType.MESH,
            )
            rdma.start()
            rdma.wait()
            out_ref[:, :] = out_ref[:, :] + comm_ref[h + 1, :, :]

        @functools.partial(
            pl.run_scoped, second_barrier=pltpu.SemaphoreType.REGULAR
        )
        def _(second_barrier):
            for nbr in [left, right]:
                pl.semaphore_signal(
                    second_barrier, inc=1,
                    device_id=(nbr,), device_id_type=pl.DeviceIdType.MESH,
                )
            pl.semaphore_wait(second_barrier, 2)

    return pl.pallas_call(
        body,
        out_shape=jax.ShapeDtypeStruct((n_tok, d_ff), jnp.float32),
        in_specs=[
            pl.BlockSpec(memory_space=pltpu.VMEM),
            pl.BlockSpec(memory_space=pltpu.VMEM),
            pl.BlockSpec(memory_space=pltpu.VMEM),
        ],
        out_specs=pl.BlockSpec(memory_space=pltpu.VMEM),
        scratch_shapes=[
            pltpu.VMEM((N_DEV, n_tok, d_ff), jnp.float32),
            pltpu.SemaphoreType.DMA((N_DEV - 1,)),
            pltpu.SemaphoreType.DMA((N_DEV - 1,)),
        ],
        compiler_params=pltpu.CompilerParams(collective_id=0),
    )(x, route_idx, expert_W)


# baseline (device time: 168540 ns/iter reference)
import functools

import jax
import jax.numpy as jnp
from jax import lax
from jax.experimental import pallas as pl
from jax.experimental.pallas import tpu as pltpu

N_DEV = 4


def kernel(x, router_W, route_idx, expert_W):
    del router_W
    n_tok, d_model = x.shape
    e_local, _, d_ff = expert_W.shape
    blk = n_tok // N_DEV

    def body(x_ref, idx_ref, w_ref, out_ref, stage_ref,
             rs_send, rs_recv, ag_send, ag_recv):
        my_pos = lax.axis_index("i")
        left = (my_pos - 1) % N_DEV
        right = (my_pos + 1) % N_DEV

        def rows(b):
            return pl.ds(b * blk, blk)

        route = idx_ref[:, :]
        for el in range(e_local):
            ge = my_pos * e_local + el
            mask = (route == ge).astype(jnp.float32)
            xm = x_ref[:, :] * mask
            part = jnp.dot(xm, w_ref[el], preferred_element_type=jnp.float32)
            if el == 0:
                out_ref[:, :] = part
            else:
                out_ref[:, :] = out_ref[:, :] + part

        barrier_sem = pltpu.get_barrier_semaphore()
        for nbr in [left, right]:
            pl.semaphore_signal(
                barrier_sem, inc=1,
                device_id=(nbr,), device_id_type=pl.DeviceIdType.MESH,
            )
        pl.semaphore_wait(barrier_sem, 2)

        for s in range(N_DEV - 1):
            send_b = (my_pos - s) % N_DEV
            recv_b = (my_pos - 1 - s) % N_DEV
            rdma = pltpu.make_async_remote_copy(
                src_ref=out_ref.at[rows(send_b), :],
                dst_ref=stage_ref.at[s],
                send_sem=rs_send.at[s],
                recv_sem=rs_recv.at[s],
                device_id=(right,),
                device_id_type=pl.DeviceIdType.MESH,
            )
            rdma.start()
            rdma.wait()
            out_ref[rows(recv_b), :] = (
                out_ref[rows(recv_b), :] + stage_ref[s, :, :]
            )

        for s in range(N_DEV - 1):
            send_b = (my_pos + 1 - s) % N_DEV
            recv_b = (my_pos - s) % N_DEV
            rdma = pltpu.make_async_remote_copy(
                src_ref=out_ref.at[rows(send_b), :],
                dst_ref=out_ref.at[rows(send_b), :],
                send_sem=ag_send.at[s],
                recv_sem=ag_recv.at[s],
                device_id=(right,),
                device_id_type=pl.DeviceIdType.MESH,
            )
            rdma.start()
            rdma.wait()
            del recv_b

        @functools.partial(
            pl.run_scoped, second_barrier=pltpu.SemaphoreType.REGULAR
        )
        def _(second_barrier):
            for nbr in [left, right]:
                pl.semaphore_signal(
                    second_barrier, inc=1,
                    device_id=(nbr,), device_id_type=pl.DeviceIdType.MESH,
                )
            pl.semaphore_wait(second_barrier, 2)

    return pl.pallas_call(
        body,
        out_shape=jax.ShapeDtypeStruct((n_tok, d_ff), jnp.float32),
        in_specs=[
            pl.BlockSpec(memory_space=pltpu.VMEM),
            pl.BlockSpec(memory_space=pltpu.VMEM),
            pl.BlockSpec(memory_space=pltpu.VMEM),
        ],
        out_specs=pl.BlockSpec(memory_space=pltpu.VMEM),
        scratch_shapes=[
            pltpu.VMEM((N_DEV - 1, blk, d_ff), jnp.float32),
            pltpu.SemaphoreType.DMA((N_DEV - 1,)),
            pltpu.SemaphoreType.DMA((N_DEV - 1,)),
            pltpu.SemaphoreType.DMA((N_DEV - 1,)),
            pltpu.SemaphoreType.DMA((N_DEV - 1,)),
        ],
        compiler_params=pltpu.CompilerParams(collective_id=0),
    )(x, route_idx, expert_W)


# device time: 94468 ns/iter; 1.7841x vs baseline; 1.7841x over previous
import functools

import jax
import jax.numpy as jnp
from jax import lax
from jax.experimental import pallas as pl
from jax.experimental.pallas import tpu as pltpu

N_DEV = 4


def kernel(x, router_W, route_idx, expert_W):
    del router_W
    n_tok, d_model = x.shape
    e_local, _, d_ff = expert_W.shape
    blk = n_tok // N_DEV
    hw = d_ff // 2

    def body(x_ref, idx_ref, w_ref, out_ref, stage_cw, stage_ccw,
             rs_send_cw, rs_recv_cw, rs_send_ccw, rs_recv_ccw,
             ag_send_cw, ag_recv_cw, ag_send_ccw, ag_recv_ccw):
        my_pos = lax.axis_index("i")
        left = (my_pos - 1) % N_DEV
        right = (my_pos + 1) % N_DEV

        def rows(b):
            return pl.ds((b % N_DEV) * blk, blk)

        cw_cols = pl.ds(0, hw)
        ccw_cols = pl.ds(hw, hw)

        def compute_block(b):
            r = rows(b)
            xb = x_ref[r, :]
            routeb = idx_ref[r, :]
            acc = jnp.zeros((blk, d_ff), jnp.float32)
            for el in range(e_local):
                ge = my_pos * e_local + el
                mask = (routeb == ge).astype(jnp.float32)
                acc = acc + jnp.dot(
                    xb * mask, w_ref[el], preferred_element_type=jnp.float32
                )
            out_ref[r, :] = acc

        def rs_step(s):
            cw = pltpu.make_async_remote_copy(
                src_ref=out_ref.at[rows(my_pos - s), cw_cols],
                dst_ref=stage_cw.at[s],
                send_sem=rs_send_cw.at[s],
                recv_sem=rs_recv_cw.at[s],
                device_id=(right,),
                device_id_type=pl.DeviceIdType.MESH,
            )
            ccw = pltpu.make_async_remote_copy(
                src_ref=out_ref.at[rows(my_pos + s), ccw_cols],
                dst_ref=stage_ccw.at[s],
                send_sem=rs_send_ccw.at[s],
                recv_sem=rs_recv_ccw.at[s],
                device_id=(left,),
                device_id_type=pl.DeviceIdType.MESH,
            )
            cw.start()
            ccw.start()
            return cw, ccw

        def rs_finish(s, cw, ccw):
            cw.wait()
            ccw.wait()
            r_cw = rows(my_pos - 1 - s)
            out_ref[r_cw, cw_cols] = out_ref[r_cw, cw_cols] + stage_cw[s]
            r_ccw = rows(my_pos + 1 + s)
            out_ref[r_ccw, ccw_cols] = out_ref[r_ccw, ccw_cols] + stage_ccw[s]

        compute_block(my_pos)

        barrier_sem = pltpu.get_barrier_semaphore()
        for nbr in [left, right]:
            pl.semaphore_signal(
                barrier_sem, inc=1,
                device_id=(nbr,), device_id_type=pl.DeviceIdType.MESH,
            )
        pl.semaphore_wait(barrier_sem, 2)

        cw0, ccw0 = rs_step(0)
        compute_block(my_pos - 1)
        compute_block(my_pos + 1)
        rs_finish(0, cw0, ccw0)

        cw1, ccw1 = rs_step(1)
        compute_block(my_pos + 2)
        rs_finish(1, cw1, ccw1)

        cw2, ccw2 = rs_step(2)
        rs_finish(2, cw2, ccw2)

        for s in range(N_DEV - 1):
            cw = pltpu.make_async_remote_copy(
                src_ref=out_ref.at[rows(my_pos + 1 - s), cw_cols],
                dst_ref=out_ref.at[rows(my_pos + 1 - s), cw_cols],
                send_sem=ag_send_cw.at[s],
                recv_sem=ag_recv_cw.at[s],
                device_id=(right,),
                device_id_type=pl.DeviceIdType.MESH,
            )
            ccw = pltpu.make_async_remote_copy(
                src_ref=out_ref.at[rows(my_pos - 1 + s), ccw_cols],
                dst_ref=out_ref.at[rows(my_pos - 1 + s), ccw_cols],
                send_sem=ag_send_ccw.at[s],
                recv_sem=ag_recv_ccw.at[s],
                device_id=(left,),
                device_id_type=pl.DeviceIdType.MESH,
            )
            cw.start()
            ccw.start()
            cw.wait()
            ccw.wait()

        @functools.partial(
            pl.run_scoped, second_barrier=pltpu.SemaphoreType.REGULAR
        )
        def _(second_barrier):
            for nbr in [left, right]:
                pl.semaphore_signal(
                    second_barrier, inc=1,
                    device_id=(nbr,), device_id_type=pl.DeviceIdType.MESH,
                )
            pl.semaphore_wait(second_barrier, 2)

    return pl.pallas_call(
        body,
        out_shape=jax.ShapeDtypeStruct((n_tok, d_ff), jnp.float32),
        in_specs=[
            pl.BlockSpec(memory_space=pltpu.VMEM),
            pl.BlockSpec(memory_space=pltpu.VMEM),
            pl.BlockSpec(memory_space=pltpu.VMEM),
        ],
        out_specs=pl.BlockSpec(memory_space=pltpu.VMEM),
        scratch_shapes=[
            pltpu.VMEM((N_DEV - 1, blk, hw), jnp.float32),
            pltpu.VMEM((N_DEV - 1, blk, hw), jnp.float32),
            pltpu.SemaphoreType.DMA((N_DEV - 1,)),
            pltpu.SemaphoreType.DMA((N_DEV - 1,)),
            pltpu.SemaphoreType.DMA((N_DEV - 1,)),
            pltpu.SemaphoreType.DMA((N_DEV - 1,)),
            pltpu.SemaphoreType.DMA((N_DEV - 1,)),
            pltpu.SemaphoreType.DMA((N_DEV - 1,)),
            pltpu.SemaphoreType.DMA((N_DEV - 1,)),
            pltpu.SemaphoreType.DMA((N_DEV - 1,)),
        ],
        compiler_params=pltpu.CompilerParams(collective_id=0),
    )(x, route_idx, expert_W)
